# baseline (speedup 1.0000x reference)
import functools

import jax
import jax.numpy as jnp
from jax import lax
from jax.experimental import pallas as pl
from jax.experimental.pallas import tpu as pltpu
from jax.experimental.pallas import tpu_sc as plsc

E = 128
C = 5
B = 16384
NC = 2
NS = 16
NW = NC * NS
BPW = B // NW
CB = 32
NCHUNK = BPW // CB
NBUF = 4
LANES = 8


def _w2v_body(tgt_hbm, ctx_hbm, ttab_hbm, ctab_hbm, out_hbm,
              tidx_v, cidx_v, wbuf, cbuf, out_v, *sems):
    wid = lax.axis_index("s") * NC + lax.axis_index("c")
    base = wid * BPW

    pltpu.sync_copy(tgt_hbm.at[pl.ds(base, BPW)], tidx_v)
    pltpu.sync_copy(ctx_hbm.at[:, pl.ds(base, BPW)], cidx_v)

    def start(k, slot):
        sw, sc = sems[2 * slot], sems[2 * slot + 1]
        koff = pl.multiple_of(k * CB, 8)
        pltpu.make_async_copy(
            ttab_hbm.at[tidx_v.at[pl.ds(koff, CB)]], wbuf.at[slot], sw
        ).start()
        for c in range(C):
            pltpu.make_async_copy(
                ctab_hbm.at[cidx_v.at[c, pl.ds(koff, CB)]],
                cbuf.at[slot, pl.ds(c * CB, CB)], sc
            ).start()

    def wait(slot):
        sw, sc = sems[2 * slot], sems[2 * slot + 1]
        pltpu.make_async_copy(ttab_hbm.at[pl.ds(0, CB)], wbuf.at[slot], sw).wait()
        for c in range(C):
            pltpu.make_async_copy(
                ttab_hbm.at[pl.ds(0, CB)], cbuf.at[slot, pl.ds(c * CB, CB)], sc
            ).wait()

    lane = lax.iota(jnp.int32, 16)

    def compute(k, slot):
        for h in range(CB // 16):
            def row_body(jj, vecs, h=h):
                j = h * 16 + jj
                w = [wbuf[slot, j, pl.ds(16 * t, 16)] for t in range(LANES)]
                out = []
                for c in range(C):
                    r = c * CB + j
                    acc = w[0] * cbuf[slot, r, pl.ds(0, 16)]
                    for t in range(1, LANES):
                        acc = acc + w[t] * cbuf[slot, r, pl.ds(16 * t, 16)]
                    out.append(jnp.where(lane == jj, jnp.sum(acc), vecs[c]))
                return tuple(out)

            init = tuple(jnp.zeros((16,), jnp.float32) for _ in range(C))
            vecs = lax.fori_loop(0, 16, row_body, init)
            koff = pl.multiple_of(k * CB + h * 16, 16)
            for c in range(C):
                out_v[c, pl.ds(koff, 16)] = vecs[c]

    for s in range(NBUF - 1):
        start(s, s)

    def group_body(g, _):
        for b in range(NBUF):
            k = g * NBUF + b

            @pl.when(k + NBUF - 1 < NCHUNK)
            def _prefetch():
                start(k + NBUF - 1, (b + NBUF - 1) % NBUF)

            wait(b)
            compute(k, b)
        return _

    lax.fori_loop(0, NCHUNK // NBUF, group_body, None)
    pltpu.sync_copy(out_v, out_hbm.at[:, pl.ds(base, BPW)])


@jax.jit
def _w2v(tgt, ctx_t, ttab, ctab):
    mesh = plsc.VectorSubcoreMesh(core_axis_name="c", subcore_axis_name="s")
    f = functools.partial(
        pl.kernel,
        out_type=jax.ShapeDtypeStruct((C, B), jnp.float32),
        mesh=mesh,
        compiler_params=pltpu.CompilerParams(needs_layout_passes=False),
        scratch_types=[
            pltpu.VMEM((BPW,), jnp.int32),
            pltpu.VMEM((C, BPW), jnp.int32),
            pltpu.VMEM((NBUF, CB, E), jnp.float32),
            pltpu.VMEM((NBUF, CB * C, E), jnp.float32),
            pltpu.VMEM((C, BPW), jnp.float32),
        ] + [pltpu.SemaphoreType.DMA] * (2 * NBUF),
    )(_w2v_body)
    return f(tgt, ctx_t, ttab, ctab)


def kernel(target, context, target_table, context_table):
    if target.ndim == 2:
        target = jnp.squeeze(target, axis=1)
    out_t = _w2v(target, context.T, target_table, context_table)
    return out_t.T


# speedup vs baseline: 1.0501x; 1.0501x over previous
import functools

import jax
import jax.numpy as jnp
from jax import lax
from jax.experimental import pallas as pl
from jax.experimental.pallas import tpu as pltpu
from jax.experimental.pallas import tpu_sc as plsc

E = 128
C = 5
B = 16384
NC = 2
NS = 16
NW = NC * NS
BPW = B // NW
CB = 16
NCHUNK = BPW // CB
NBUF = 4
LANES = 8


def _w2v_body(tgt_hbm, ctx_hbm, ttab_hbm, ctab_hbm, out_hbm,
              tidx_v, cidx_v, wbuf, cbuf, out_v, *sems):
    wid = lax.axis_index("s") * NC + lax.axis_index("c")
    base = wid * BPW

    pltpu.sync_copy(tgt_hbm.at[pl.ds(base, BPW)], tidx_v)
    pltpu.sync_copy(ctx_hbm.at[:, pl.ds(base, BPW)], cidx_v)

    def start(k, slot):
        sw, sc = sems[2 * slot], sems[2 * slot + 1]
        koff = pl.multiple_of(k * CB, 8)
        pltpu.make_async_copy(
            ttab_hbm.at[tidx_v.at[pl.ds(koff, CB)]], wbuf.at[slot], sw
        ).start()
        for c in range(C):
            pltpu.make_async_copy(
                ctab_hbm.at[cidx_v.at[c, pl.ds(koff, CB)]],
                cbuf.at[slot, pl.ds(c * CB, CB)], sc
            ).start()

    def wait(slot):
        sw, sc = sems[2 * slot], sems[2 * slot + 1]
        pltpu.make_async_copy(ttab_hbm.at[pl.ds(0, CB)], wbuf.at[slot], sw).wait()
        for c in range(C):
            pltpu.make_async_copy(
                ttab_hbm.at[pl.ds(0, CB)], cbuf.at[slot, pl.ds(c * CB, CB)], sc
            ).wait()

    lane = lax.iota(jnp.int32, 16)

    def compute(k, slot):
        def row_body(j, vecs):
            w = [wbuf[slot, j, pl.ds(16 * t, 16)] for t in range(LANES)]
            out = []
            for c in range(C):
                r = c * CB + j
                acc = w[0] * cbuf[slot, r, pl.ds(0, 16)]
                for t in range(1, LANES):
                    acc = acc + w[t] * cbuf[slot, r, pl.ds(16 * t, 16)]
                out.append(jnp.where(lane == j, jnp.sum(acc), vecs[c]))
            return tuple(out)

        init = tuple(jnp.zeros((16,), jnp.float32) for _ in range(C))
        vecs = lax.fori_loop(0, CB, row_body, init)
        koff = pl.multiple_of(k * CB, 16)
        for c in range(C):
            out_v[c, pl.ds(koff, 16)] = vecs[c]

    for s in range(NBUF - 1):
        start(s, s)

    def group_body(g, _):
        for b in range(NBUF):
            k = g * NBUF + b

            @pl.when(k + NBUF - 1 < NCHUNK)
            def _prefetch():
                start(k + NBUF - 1, (b + NBUF - 1) % NBUF)

            wait(b)
            compute(k, b)
        return _

    lax.fori_loop(0, NCHUNK // NBUF, group_body, None)
    pltpu.sync_copy(out_v, out_hbm.at[:, pl.ds(base, BPW)])


@jax.jit
def _w2v(tgt, ctx_t, ttab, ctab):
    mesh = plsc.VectorSubcoreMesh(core_axis_name="c", subcore_axis_name="s")
    f = functools.partial(
        pl.kernel,
        out_type=jax.ShapeDtypeStruct((C, B), jnp.float32),
        mesh=mesh,
        compiler_params=pltpu.CompilerParams(needs_layout_passes=False),
        scratch_types=[
            pltpu.VMEM((BPW,), jnp.int32),
            pltpu.VMEM((C, BPW), jnp.int32),
            pltpu.VMEM((NBUF, CB, E), jnp.float32),
            pltpu.VMEM((NBUF, CB * C, E), jnp.float32),
            pltpu.VMEM((C, BPW), jnp.float32),
        ] + [pltpu.SemaphoreType.DMA] * (2 * NBUF),
    )(_w2v_body)
    return f(tgt, ctx_t, ttab, ctab)


def kernel(target, context, target_table, context_table):
    if target.ndim == 2:
        target = jnp.squeeze(target, axis=1)
    out_t = _w2v(target, context.T, target_table, context_table)
    return out_t.T
